# Initial kernel scaffold; baseline (speedup 1.0000x reference)
import jax
import jax.numpy as jnp
from jax.experimental import pallas as pl


def kernel(input_ids, token_table, position_table):
    raise NotImplementedError("write your pallas kernel here")


# speedup vs baseline: 1.5097x; 1.5097x over previous
import functools

import jax
import jax.numpy as jnp
from jax import lax
from jax.experimental import pallas as pl
from jax.experimental.pallas import tpu as pltpu
from jax.experimental.pallas import tpu_sc as plsc

VOCAB = 100000
D = 768
B = 4
S = 2048
NC = 2
NS = 16
NW = NC * NS
RPW = S // NW
LANES = 16
VECS_PER_ROW = D // LANES


def _body(ids_hbm, tok_hbm, pos_hbm, out_hbm, idx_v, pos_v, tok_v, sem):
    wid = lax.axis_index("s") * NC + lax.axis_index("c")
    base = wid * RPW

    pltpu.sync_copy(pos_hbm.at[pl.ds(base, RPW)], pos_v)

    for b in range(B):
        row0 = b * S + base
        pltpu.sync_copy(ids_hbm.at[pl.ds(row0, RPW)], idx_v)
        pltpu.async_copy(tok_hbm.at[idx_v], tok_v, sem).wait()

        def add_row(r, carry):
            tv = tok_v.at[r]
            pv = pos_v.at[r]
            for j in range(VECS_PER_ROW):
                sl = pl.ds(j * LANES, LANES)
                tv[sl] = tv[sl] + pv[sl]
            return carry

        lax.fori_loop(0, RPW, add_row, 0)
        pltpu.sync_copy(tok_v, out_hbm.at[pl.ds(row0, RPW)])


@functools.partial(jax.jit, static_argnames=())
def _embed(ids_flat, token_table, position_table):
    mesh = plsc.VectorSubcoreMesh(core_axis_name="c", subcore_axis_name="s")
    run = pl.kernel(
        _body,
        out_type=jax.ShapeDtypeStruct((B * S, D), jnp.float32),
        mesh=mesh,
        scratch_types=[
            pltpu.VMEM((RPW,), jnp.int32),
            pltpu.VMEM((RPW, D), jnp.float32),
            pltpu.VMEM((RPW, D), jnp.float32),
            pltpu.SemaphoreType.DMA,
        ],
    )
    return run(ids_flat, token_table, position_table)


def kernel(input_ids, token_table, position_table):
    ids_flat = input_ids.reshape(-1).astype(jnp.int32)
    out = _embed(ids_flat, token_table, position_table)
    return out.reshape(B, S, D)
